# baseline (device time: 1206525 ns/iter reference)
import os

import jax
import jax.numpy as jnp
from jax import lax
from jax.experimental import pallas as pl
from jax.experimental.pallas import tpu as pltpu

os.makedirs("/tmp/jax_cache", exist_ok=True)
jax.config.update("jax_compilation_cache_dir", "/tmp/jax_cache")
jax.config.update("jax_persistent_cache_min_compile_time_secs", 1.0)

N_DEV = 32
M_PER = 128
WIRE_SCALE = 8192.0
NSLOT = 6
LAG = 2


def kernel(x, w_mat):
    m_total, k_per = x.shape
    _, n_cols = w_mat.shape

    def body(x_ref, w_ref, out_ref, land_ref,
             stage, rstage, send_sems, recv_sems, copy_sems,
             amax_buf, amax_send_sems, amax_recv_sems):
        my = lax.axis_index("i")

        barrier_sem = pltpu.get_barrier_semaphore()
        for off in range(1, N_DEV):
            pl.semaphore_signal(barrier_sem, inc=1,
                                device_id=((my + off) % N_DEV,),
                                device_id_type=pl.DeviceIdType.MESH)
        pl.semaphore_wait(barrier_sem, N_DEV - 1)

        def partial_for(chunk):
            xc = x_ref[pl.ds(chunk * M_PER, M_PER), :]
            return jnp.dot(xc, w_ref[...],
                           preferred_element_type=jnp.float32,
                           precision=lax.Precision.HIGHEST)

        out_ref[...] = partial_for(my)

        def make_flow(off, slot):
            return pltpu.make_async_remote_copy(
                src_ref=stage.at[slot],
                dst_ref=land_ref.at[off - 1],
                send_sem=send_sems.at[slot],
                recv_sem=recv_sems.at[off - 1],
                device_id=((my + off) % N_DEV,),
                device_id_type=pl.DeviceIdType.MESH,
            )

        pending = [None] * NSLOT

        def process_recv(off):
            make_flow(off, (off - 1) % NSLOT).wait_recv()
            cslot = off % 2
            cp = pltpu.make_async_copy(
                land_ref.at[off - 1], rstage.at[cslot], copy_sems.at[cslot])
            cp.start()
            cp.wait()
            out_ref[...] = out_ref[...] + (
                rstage[cslot].astype(jnp.float32) * (1.0 / WIRE_SCALE))

        for off in range(1, N_DEV):
            slot = (off - 1) % NSLOT
            if pending[slot] is not None:
                pending[slot].wait_send()
            p = partial_for((my + off) % N_DEV)
            stage[slot] = jnp.round(
                jnp.clip(p * WIRE_SCALE, -32704.0, 32704.0)
            ).astype(jnp.int16)
            d = make_flow(off, slot)
            d.start()
            pending[slot] = d
            if off > LAG:
                process_recv(off - LAG)
        for off in range(N_DEV - LAG, N_DEV):
            process_recv(off)
        for slot in range(NSLOT):
            if pending[slot] is not None:
                pending[slot].wait_send()

        local_amax = jnp.max(jnp.abs(out_ref[...]))
        amax_buf[N_DEV - 1] = jnp.full((8, 128), local_amax, jnp.float32)
        descs = []
        for off in range(1, N_DEV):
            tgt = (my + off) % N_DEV
            d = pltpu.make_async_remote_copy(
                src_ref=amax_buf.at[N_DEV - 1],
                dst_ref=amax_buf.at[off - 1],
                send_sem=amax_send_sems.at[off - 1],
                recv_sem=amax_recv_sems.at[off - 1],
                device_id=(tgt,),
                device_id_type=pl.DeviceIdType.MESH,
            )
            d.start()
            descs.append(d)
        for d in descs:
            d.wait_send()
        for d in descs:
            d.wait_recv()
        gmax = jnp.max(amax_buf[...])

        scale = gmax / 448.0
        q = (out_ref[...] / scale).astype(jnp.float8_e4m3fn)
        out_ref[...] = q.astype(jnp.float32) * scale

    out, _ = pl.pallas_call(
        body,
        out_shape=(
            jax.ShapeDtypeStruct((M_PER, n_cols), jnp.float32),
            jax.ShapeDtypeStruct((N_DEV - 1, M_PER, n_cols), jnp.int16),
        ),
        in_specs=[
            pl.BlockSpec(memory_space=pltpu.VMEM),
            pl.BlockSpec(memory_space=pltpu.VMEM),
        ],
        out_specs=(
            pl.BlockSpec(memory_space=pltpu.VMEM),
            pl.BlockSpec(memory_space=pltpu.HBM),
        ),
        scratch_shapes=[
            pltpu.VMEM((NSLOT, M_PER, n_cols), jnp.int16),
            pltpu.VMEM((2, M_PER, n_cols), jnp.int16),
            pltpu.SemaphoreType.DMA((NSLOT,)),
            pltpu.SemaphoreType.DMA((N_DEV - 1,)),
            pltpu.SemaphoreType.DMA((2,)),
            pltpu.VMEM((N_DEV, 8, 128), jnp.float32),
            pltpu.SemaphoreType.DMA((N_DEV - 1,)),
            pltpu.SemaphoreType.DMA((N_DEV - 1,)),
        ],
        compiler_params=pltpu.CompilerParams(collective_id=0),
    )(x, w_mat)
    return out


# device time: 747745 ns/iter; 1.6136x vs baseline; 1.6136x over previous
import os

import jax
import jax.numpy as jnp
from jax import lax
from jax.experimental import pallas as pl
from jax.experimental.pallas import tpu as pltpu

os.makedirs("/tmp/jax_cache", exist_ok=True)
jax.config.update("jax_compilation_cache_dir", "/tmp/jax_cache")
jax.config.update("jax_persistent_cache_min_compile_time_secs", 1.0)

N_DEV = 32
M_PER = 128
N_LANE = 4
WIRE_SCALE = 4096.0


def kernel(x, w_mat):
    m_total, k_per = x.shape
    _, n_cols = w_mat.shape
    half = n_cols // 2
    quart = n_cols // N_LANE

    def body(x_ref, w_ref, out_ref, *rest):
        sbufs = rest[0:8:2]
        rbufs = rest[1:8:2]
        send_sems = rest[8:16:2]
        recv_sems = rest[9:16:2]
        credits = rest[16:20]
        amax_buf, amax_send_sems, amax_recv_sems = rest[20:23]

        my = lax.axis_index("i")
        left = (my - 1) % N_DEV
        right = (my + 1) % N_DEV

        barrier_sem = pltpu.get_barrier_semaphore()
        for nbr in (left, right):
            pl.semaphore_signal(barrier_sem, inc=1, device_id=(nbr,),
                                device_id_type=pl.DeviceIdType.MESH)
        pl.semaphore_wait(barrier_sem, 2)

        dsts = (right, right, left, left)
        credit_dsts = (left, left, right, right)

        def make_rdma(li, slot):
            return pltpu.make_async_remote_copy(
                src_ref=sbufs[li].at[slot], dst_ref=rbufs[li].at[slot],
                send_sem=send_sems[li].at[slot],
                recv_sem=recv_sems[li].at[slot],
                device_id=(dsts[li],), device_id_type=pl.DeviceIdType.MESH,
            )

        pending = [[None, None] for _ in range(N_LANE)]

        for s in range(N_DEV):
            chunks = ((my - 1 - s) % N_DEV, (my + 1 + s) % N_DEV)
            partials = []
            for di in range(2):
                xc = x_ref[pl.ds(chunks[di] * M_PER, M_PER), :]
                wc = w_ref[:, pl.ds(di * half, half)]
                partials.append(jnp.dot(
                    xc, wc, preferred_element_type=jnp.float32,
                    precision=lax.Precision.HIGHEST))
            for li in range(N_LANE):
                lo = (li % 2) * quart
                pq = partials[li // 2][:, lo:lo + quart]
                if s == 0:
                    val = pq
                else:
                    r = (s - 1) % 2
                    make_rdma(li, r).wait_recv()
                    val = pq + rbufs[li][r].astype(jnp.float32) * (
                        1.0 / WIRE_SCALE)
                    pl.semaphore_signal(credits[li].at[r], inc=1,
                                        device_id=(credit_dsts[li],),
                                        device_id_type=pl.DeviceIdType.MESH)
                if s < N_DEV - 1:
                    k = s % 2
                    if pending[li][k] is not None:
                        pending[li][k].wait_send()
                    sbufs[li][k] = jnp.round(
                        jnp.clip(val * WIRE_SCALE, -32704.0, 32704.0)
                    ).astype(jnp.int16)
                    if s >= 2:
                        pl.semaphore_wait(credits[li].at[k], 1)
                    d = make_rdma(li, k)
                    d.start()
                    pending[li][k] = d
                else:
                    out_ref[:, pl.ds(li * quart, quart)] = val

        for li in range(N_LANE):
            for k in range(2):
                if pending[li][k] is not None:
                    pending[li][k].wait_send()
            pl.semaphore_wait(credits[li].at[0], 1)
            pl.semaphore_wait(credits[li].at[1], 1)

        local_amax = jnp.max(jnp.abs(out_ref[...]))
        amax_buf[N_DEV - 1] = jnp.full((8, 128), local_amax, jnp.float32)
        descs = []
        for off in range(1, N_DEV):
            tgt = (my + off) % N_DEV
            d = pltpu.make_async_remote_copy(
                src_ref=amax_buf.at[N_DEV - 1],
                dst_ref=amax_buf.at[off - 1],
                send_sem=amax_send_sems.at[off - 1],
                recv_sem=amax_recv_sems.at[off - 1],
                device_id=(tgt,),
                device_id_type=pl.DeviceIdType.MESH,
            )
            d.start()
            descs.append(d)
        for d in descs:
            d.wait_send()
        for d in descs:
            d.wait_recv()
        gmax = jnp.max(amax_buf[...])

        scale = gmax / 448.0
        q = (out_ref[...] / scale).astype(jnp.float8_e4m3fn)
        out_ref[...] = q.astype(jnp.float32) * scale

    scratch = []
    for _ in range(N_LANE):
        scratch.append(pltpu.VMEM((2, M_PER, quart), jnp.int16))
        scratch.append(pltpu.VMEM((2, M_PER, quart), jnp.int16))
    for _ in range(N_LANE):
        scratch.append(pltpu.SemaphoreType.DMA((2,)))
        scratch.append(pltpu.SemaphoreType.DMA((2,)))
    for _ in range(N_LANE):
        scratch.append(pltpu.SemaphoreType.REGULAR((2,)))
    scratch += [
        pltpu.VMEM((N_DEV, 8, 128), jnp.float32),
        pltpu.SemaphoreType.DMA((N_DEV - 1,)),
        pltpu.SemaphoreType.DMA((N_DEV - 1,)),
    ]

    return pl.pallas_call(
        body,
        out_shape=jax.ShapeDtypeStruct((M_PER, n_cols), jnp.float32),
        in_specs=[
            pl.BlockSpec(memory_space=pltpu.VMEM),
            pl.BlockSpec(memory_space=pltpu.VMEM),
        ],
        out_specs=pl.BlockSpec(memory_space=pltpu.VMEM),
        scratch_shapes=scratch,
        compiler_params=pltpu.CompilerParams(collective_id=0),
    )(x, w_mat)


# device time: 730102 ns/iter; 1.6525x vs baseline; 1.0242x over previous
import os

import jax
import jax.numpy as jnp
from jax import lax
from jax.experimental import pallas as pl
from jax.experimental.pallas import tpu as pltpu

os.makedirs("/tmp/jax_cache", exist_ok=True)
jax.config.update("jax_compilation_cache_dir", "/tmp/jax_cache")
jax.config.update("jax_persistent_cache_min_compile_time_secs", 1.0)

N_DEV = 32
M_PER = 128
N_LANE = 8
WIRE_SCALE = 4096.0


def kernel(x, w_mat):
    m_total, k_per = x.shape
    _, n_cols = w_mat.shape
    half = n_cols // 2
    quart = n_cols // N_LANE

    def body(x_ref, w_ref, out_ref, *rest):
        nb = 2 * N_LANE
        sbufs = rest[0:nb:2]
        rbufs = rest[1:nb:2]
        send_sems = rest[nb:2 * nb:2]
        recv_sems = rest[nb + 1:2 * nb:2]
        credits = rest[2 * nb:2 * nb + N_LANE]
        amax_buf, amax_send_sems, amax_recv_sems = rest[2 * nb + N_LANE:]

        my = lax.axis_index("i")
        left = (my - 1) % N_DEV
        right = (my + 1) % N_DEV

        barrier_sem = pltpu.get_barrier_semaphore()
        for nbr in (left, right):
            pl.semaphore_signal(barrier_sem, inc=1, device_id=(nbr,),
                                device_id_type=pl.DeviceIdType.MESH)
        pl.semaphore_wait(barrier_sem, 2)

        dsts = tuple(right if li < N_LANE // 2 else left
                     for li in range(N_LANE))
        credit_dsts = tuple(left if li < N_LANE // 2 else right
                            for li in range(N_LANE))

        def make_rdma(li, slot):
            return pltpu.make_async_remote_copy(
                src_ref=sbufs[li].at[slot], dst_ref=rbufs[li].at[slot],
                send_sem=send_sems[li].at[slot],
                recv_sem=recv_sems[li].at[slot],
                device_id=(dsts[li],), device_id_type=pl.DeviceIdType.MESH,
            )

        pending = [[None, None] for _ in range(N_LANE)]

        for s in range(N_DEV):
            chunks = ((my - 1 - s) % N_DEV, (my + 1 + s) % N_DEV)
            partials = []
            for di in range(2):
                xc = x_ref[pl.ds(chunks[di] * M_PER, M_PER), :]
                wc = w_ref[:, pl.ds(di * half, half)]
                partials.append(jnp.dot(
                    xc, wc, preferred_element_type=jnp.float32,
                    precision=lax.Precision.HIGHEST))
            for li in range(N_LANE):
                lo = (li % (N_LANE // 2)) * quart
                pq = partials[li // (N_LANE // 2)][:, lo:lo + quart]
                if s == 0:
                    val = pq
                else:
                    r = (s - 1) % 2
                    make_rdma(li, r).wait_recv()
                    val = pq + rbufs[li][r].astype(jnp.float32) * (
                        1.0 / WIRE_SCALE)
                    pl.semaphore_signal(credits[li].at[r], inc=1,
                                        device_id=(credit_dsts[li],),
                                        device_id_type=pl.DeviceIdType.MESH)
                if s < N_DEV - 1:
                    k = s % 2
                    if pending[li][k] is not None:
                        pending[li][k].wait_send()
                    sbufs[li][k] = jnp.round(
                        jnp.clip(val * WIRE_SCALE, -32704.0, 32704.0)
                    ).astype(jnp.int16)
                    if s >= 2:
                        pl.semaphore_wait(credits[li].at[k], 1)
                    d = make_rdma(li, k)
                    d.start()
                    pending[li][k] = d
                else:
                    out_ref[:, pl.ds(li * quart, quart)] = val

        for li in range(N_LANE):
            for k in range(2):
                if pending[li][k] is not None:
                    pending[li][k].wait_send()
            pl.semaphore_wait(credits[li].at[0], 1)
            pl.semaphore_wait(credits[li].at[1], 1)

        local_amax = jnp.max(jnp.abs(out_ref[...]))
        amax_buf[N_DEV - 1] = jnp.full((8, 128), local_amax, jnp.float32)
        descs = []
        for off in range(1, N_DEV):
            tgt = (my + off) % N_DEV
            d = pltpu.make_async_remote_copy(
                src_ref=amax_buf.at[N_DEV - 1],
                dst_ref=amax_buf.at[off - 1],
                send_sem=amax_send_sems.at[off - 1],
                recv_sem=amax_recv_sems.at[off - 1],
                device_id=(tgt,),
                device_id_type=pl.DeviceIdType.MESH,
            )
            d.start()
            descs.append(d)
        for d in descs:
            d.wait_send()
        for d in descs:
            d.wait_recv()
        gmax = jnp.max(amax_buf[...])

        scale = gmax / 448.0
        q = (out_ref[...] / scale).astype(jnp.float8_e4m3fn)
        out_ref[...] = q.astype(jnp.float32) * scale

    scratch = []
    for _ in range(N_LANE):
        scratch.append(pltpu.VMEM((2, M_PER, quart), jnp.int16))
        scratch.append(pltpu.VMEM((2, M_PER, quart), jnp.int16))
    for _ in range(N_LANE):
        scratch.append(pltpu.SemaphoreType.DMA((2,)))
        scratch.append(pltpu.SemaphoreType.DMA((2,)))
    for _ in range(N_LANE):
        scratch.append(pltpu.SemaphoreType.REGULAR((2,)))
    scratch += [
        pltpu.VMEM((N_DEV, 8, 128), jnp.float32),
        pltpu.SemaphoreType.DMA((N_DEV - 1,)),
        pltpu.SemaphoreType.DMA((N_DEV - 1,)),
    ]

    return pl.pallas_call(
        body,
        out_shape=jax.ShapeDtypeStruct((M_PER, n_cols), jnp.float32),
        in_specs=[
            pl.BlockSpec(memory_space=pltpu.VMEM),
            pl.BlockSpec(memory_space=pltpu.VMEM),
        ],
        out_specs=pl.BlockSpec(memory_space=pltpu.VMEM),
        scratch_shapes=scratch,
        compiler_params=pltpu.CompilerParams(collective_id=0),
    )(x, w_mat)
